# Initial kernel scaffold; baseline (speedup 1.0000x reference)
import jax
import jax.numpy as jnp
from jax.experimental import pallas as pl


def kernel(x, seg, tok_table, pos_table, seg_table):
    raise NotImplementedError("write your pallas kernel here")


# speedup vs baseline: 2.9247x; 2.9247x over previous
import functools

import jax
import jax.numpy as jnp
from jax import lax
from jax.experimental import pallas as pl
from jax.experimental.pallas import tpu as pltpu
from jax.experimental.pallas import tpu_sc as plsc

VOCAB = 4
NSEG = 2
SEQ = 30
D = 768
BATCH = 4096
NTOK = BATCH * SEQ

NC = 2
NS = 16
NW = NC * NS
BPW = NTOK // NW
K = 128
NCHUNK = BPW // K


def _comb_body(tok_ref, pos_ref, seg_ref, out_ref):
    p = pos_ref[...]
    for v in range(VOCAB):
        for g in range(NSEG):
            e = p + tok_ref[v : v + 1, :] + seg_ref[g : g + 1, :]
            m = jnp.mean(e, axis=-1, keepdims=True)
            var = jnp.mean((e - m) ** 2, axis=-1, keepdims=True)
            out_ref[(v * NSEG + g) * SEQ : (v * NSEG + g + 1) * SEQ, :] = (
                e - m
            ) * lax.rsqrt(var + 1e-5)


def _build_comb(tok_table, pos_table, seg_table):
    return pl.pallas_call(
        _comb_body,
        out_shape=jax.ShapeDtypeStruct((VOCAB * NSEG * SEQ, D), jnp.float32),
    )(tok_table, pos_table, seg_table)


@functools.lru_cache(maxsize=1)
def _make_expand():
    mesh = plsc.VectorSubcoreMesh(
        core_axis_name="c", subcore_axis_name="s", num_cores=NC, num_subcores=NS
    )

    @functools.partial(
        pl.kernel,
        out_type=jax.ShapeDtypeStruct((NTOK, D), jnp.float32),
        mesh=mesh,
        scratch_types=[
            pltpu.VMEM((K,), jnp.int32),
            pltpu.VMEM((K,), jnp.int32),
            pltpu.VMEM((K,), jnp.int32),
            pltpu.VMEM((K, D), jnp.float32),
            pltpu.SemaphoreType.DMA,
        ],
    )
    def _expand(x_hbm, seg_hbm, comb_hbm, out_hbm, xv, sv, idxv, rows, sem):
        wid = lax.axis_index("s") * NC + lax.axis_index("c")
        base = wid * BPW

        def chunk(i, carry):
            gbase = base + i * K
            pltpu.sync_copy(x_hbm.at[pl.ds(gbase, K)], xv)
            pltpu.sync_copy(seg_hbm.at[pl.ds(gbase, K)], sv)
            for j in range(K // 16):
                t0 = gbase + j * 16
                tvec = t0 + lax.iota(jnp.int32, 16)
                pos = tvec % SEQ
                idxv[pl.ds(j * 16, 16)] = (
                    xv[pl.ds(j * 16, 16)] * (NSEG * SEQ)
                    + sv[pl.ds(j * 16, 16)] * SEQ
                    + pos
                )
            pltpu.async_copy(comb_hbm.at[idxv], rows, sem).wait()
            pltpu.sync_copy(rows, out_hbm.at[pl.ds(gbase, K)])
            return carry

        lax.fori_loop(0, NCHUNK, chunk, 0)

    return _expand


def kernel(x, seg, tok_table, pos_table, seg_table):
    comb = _build_comb(tok_table, pos_table, seg_table)
    xf = x.reshape(NTOK).astype(jnp.int32)
    sf = seg.reshape(NTOK).astype(jnp.int32)
    out = _make_expand()(xf, sf, comb)
    return out.reshape(BATCH, SEQ, D)


# speedup vs baseline: 2.9443x; 1.0067x over previous
import functools

import jax
import jax.numpy as jnp
from jax import lax
from jax.experimental import pallas as pl
from jax.experimental.pallas import tpu as pltpu
from jax.experimental.pallas import tpu_sc as plsc

VOCAB = 4
NSEG = 2
SEQ = 30
D = 768
BATCH = 4096
NTOK = BATCH * SEQ

NC = 2
NS = 16
NW = NC * NS
BPW = NTOK // NW
K = 64
NCHUNK = BPW // K
NPAIR = NCHUNK // 2
NROWS = VOCAB * NSEG * SEQ


def _comb_body(tok_ref, pos_ref, seg_ref, out_ref):
    p = pos_ref[...]
    for v in range(VOCAB):
        for g in range(NSEG):
            e = p + tok_ref[v : v + 1, :] + seg_ref[g : g + 1, :]
            m = jnp.mean(e, axis=-1, keepdims=True)
            var = jnp.mean((e - m) ** 2, axis=-1, keepdims=True)
            out_ref[(v * NSEG + g) * SEQ : (v * NSEG + g + 1) * SEQ, :] = (
                e - m
            ) * lax.rsqrt(var + 1e-5)


def _build_comb(tok_table, pos_table, seg_table):
    return pl.pallas_call(
        _comb_body,
        out_shape=jax.ShapeDtypeStruct((VOCAB * NSEG * SEQ, D), jnp.float32),
    )(tok_table, pos_table, seg_table)


@functools.lru_cache(maxsize=1)
def _make_expand():
    mesh = plsc.VectorSubcoreMesh(
        core_axis_name="c", subcore_axis_name="s", num_cores=NC, num_subcores=NS
    )

    @functools.partial(
        pl.kernel,
        out_type=jax.ShapeDtypeStruct((NTOK, D), jnp.float32),
        mesh=mesh,
        scratch_types=[
            pltpu.VMEM((BPW,), jnp.int32),
            pltpu.VMEM((BPW,), jnp.int32),
            pltpu.VMEM((BPW,), jnp.int32),
            pltpu.VMEM((K, D), jnp.float32),
            pltpu.VMEM((K, D), jnp.float32),
            pltpu.SemaphoreType.DMA,
            pltpu.SemaphoreType.DMA,
        ],
    )
    def _expand(
        x_hbm, seg_hbm, comb_hbm, out_hbm,
        xs, ss, idxs, rows0, rows1, semg0, semg1,
    ):
        cid = lax.axis_index("c")
        sid = lax.axis_index("s")
        wid = sid * NC + cid
        base = wid * BPW

        pltpu.sync_copy(x_hbm.at[pl.ds(base, BPW)], xs)
        pltpu.sync_copy(seg_hbm.at[pl.ds(base, BPW)], ss)

        def idx_body(j, carry):
            tvec = base + j * 16 + lax.iota(jnp.int32, 16)
            idxs[pl.ds(j * 16, 16)] = (
                xs[pl.ds(j * 16, 16)] * (NSEG * SEQ)
                + ss[pl.ds(j * 16, 16)] * SEQ
                + tvec % SEQ
            )
            return carry

        lax.fori_loop(0, BPW // 16, idx_body, 0)

        pltpu.async_copy(comb_hbm.at[idxs.at[pl.ds(0, K)]], rows0, semg0)

        def pair(g, carry):
            i0 = 2 * g
            pltpu.make_async_copy(comb_hbm.at[pl.ds(0, K)], rows0, semg0).wait()
            pltpu.async_copy(
                comb_hbm.at[idxs.at[pl.ds((i0 + 1) * K, K)]], rows1, semg1
            )
            pltpu.sync_copy(rows0, out_hbm.at[pl.ds(base + i0 * K, K)])
            pltpu.make_async_copy(comb_hbm.at[pl.ds(0, K)], rows1, semg1).wait()

            @pl.when(g < NPAIR - 1)
            def _():
                pltpu.async_copy(
                    comb_hbm.at[idxs.at[pl.ds((i0 + 2) * K, K)]], rows0, semg0
                )

            pltpu.sync_copy(rows1, out_hbm.at[pl.ds(base + (i0 + 1) * K, K)])
            return carry

        lax.fori_loop(0, NPAIR, pair, 0)

    return _expand


def kernel(x, seg, tok_table, pos_table, seg_table):
    comb = _build_comb(tok_table, pos_table, seg_table)
    xf = x.reshape(NTOK).astype(jnp.int32)
    sf = seg.reshape(NTOK).astype(jnp.int32)
    out = _make_expand()(xf, sf, comb)
    return out.reshape(BATCH, SEQ, D)


# speedup vs baseline: 5.7433x; 1.9506x over previous
import functools

import jax
import jax.numpy as jnp
from jax import lax
from jax.experimental import pallas as pl
from jax.experimental.pallas import tpu as pltpu
from jax.experimental.pallas import tpu_sc as plsc

VOCAB = 4
NSEG = 2
SEQ = 30
D = 768
BATCH = 4096
NTOK = BATCH * SEQ
NJ = VOCAB * NSEG

BB = 128


def _comb_body(tok_ref, pos_ref, seg_ref, out_ref):
    tok8 = jnp.concatenate(
        [tok_ref[v : v + 1] for v in range(VOCAB) for _ in range(NSEG)], axis=0
    )
    seg8 = jnp.concatenate(
        [seg_ref[g : g + 1] for _ in range(VOCAB) for g in range(NSEG)], axis=0
    )
    base = tok8 + seg8
    for s in range(SEQ):
        e = base + pos_ref[s : s + 1]
        m = jnp.mean(e, axis=-1, keepdims=True)
        var = jnp.mean((e - m) ** 2, axis=-1, keepdims=True)
        out_ref[s * NJ : (s + 1) * NJ] = ((e - m) * lax.rsqrt(var + 1e-5)).astype(
            jnp.bfloat16
        )


def _build_comb(tok_table, pos_table, seg_table):
    return pl.pallas_call(
        _comb_body,
        out_shape=jax.ShapeDtypeStruct((SEQ * NJ, D), jnp.bfloat16),
    )(tok_table, pos_table, seg_table)


def _expand_body(x_ref, s_ref, comb_ref, out_ref):
    xb = x_ref[...]
    sb = s_ref[...]
    jb = xb * NSEG + sb
    for s in range(SEQ):
        oh = (
            jb[:, s : s + 1]
            == lax.broadcasted_iota(jnp.int32, (BB, NJ), 1)
        ).astype(jnp.bfloat16)
        out_ref[:, s, :] = lax.dot_general(
            oh,
            comb_ref[s * NJ : (s + 1) * NJ],
            (((1,), (0,)), ((), ())),
            preferred_element_type=jnp.float32,
        )


def _expand(xi, si, comb):
    return pl.pallas_call(
        _expand_body,
        grid=(BATCH // BB,),
        in_specs=[
            pl.BlockSpec((BB, SEQ), lambda b: (b, 0)),
            pl.BlockSpec((BB, SEQ), lambda b: (b, 0)),
            pl.BlockSpec((SEQ * NJ, D), lambda b: (0, 0)),
        ],
        out_specs=pl.BlockSpec((BB, SEQ, D), lambda b: (b, 0, 0)),
        out_shape=jax.ShapeDtypeStruct((BATCH, SEQ, D), jnp.float32),
    )(xi, si, comb)


def kernel(x, seg, tok_table, pos_table, seg_table):
    comb = _build_comb(tok_table, pos_table, seg_table)
    return _expand(x.astype(jnp.int32), seg.astype(jnp.int32), comb)


# speedup vs baseline: 5.9013x; 1.0275x over previous
import functools

import jax
import jax.numpy as jnp
from jax import lax
from jax.experimental import pallas as pl
from jax.experimental.pallas import tpu as pltpu
from jax.experimental.pallas import tpu_sc as plsc

VOCAB = 4
NSEG = 2
SEQ = 30
D = 768
BATCH = 4096
NTOK = BATCH * SEQ
NJ = VOCAB * NSEG
NROWS = SEQ * NJ

BB = 128
TB = BB * SEQ


def _comb_body(tok_ref, pos_ref, seg_ref, out_ref):
    tok8 = jnp.concatenate(
        [tok_ref[v : v + 1] for v in range(VOCAB) for _ in range(NSEG)], axis=0
    )
    seg8 = jnp.concatenate(
        [seg_ref[g : g + 1] for _ in range(VOCAB) for g in range(NSEG)], axis=0
    )
    base = tok8 + seg8
    for s in range(SEQ):
        e = base + pos_ref[s : s + 1]
        m = jnp.mean(e, axis=-1, keepdims=True)
        var = jnp.mean((e - m) ** 2, axis=-1, keepdims=True)
        out_ref[s * NJ : (s + 1) * NJ] = ((e - m) * lax.rsqrt(var + 1e-5)).astype(
            jnp.bfloat16
        )


def _build_comb(tok_table, pos_table, seg_table):
    return pl.pallas_call(
        _comb_body,
        out_shape=jax.ShapeDtypeStruct((NROWS, D), jnp.bfloat16),
    )(tok_table, pos_table, seg_table)


def _expand_body(x_ref, s_ref, comb_ref, out_ref):
    xb = x_ref[...].astype(jnp.int32)
    sb = s_ref[...].astype(jnp.int32)
    pos = lax.broadcasted_iota(jnp.int32, (TB, 1), 0) % SEQ
    c = pos * NJ + xb * NSEG + sb
    oh = (c == lax.broadcasted_iota(jnp.int32, (TB, NROWS), 1)).astype(
        jnp.bfloat16
    )
    res = lax.dot_general(
        oh,
        comb_ref[...],
        (((1,), (0,)), ((), ())),
        preferred_element_type=jnp.float32,
    )
    out_ref[...] = res.reshape(BB, SEQ, D)


def _expand(xi, si, comb):
    return pl.pallas_call(
        _expand_body,
        grid=(BATCH // BB,),
        in_specs=[
            pl.BlockSpec((TB, 1), lambda b: (b, 0)),
            pl.BlockSpec((TB, 1), lambda b: (b, 0)),
            pl.BlockSpec((NROWS, D), lambda b: (0, 0)),
        ],
        out_specs=pl.BlockSpec((BB, SEQ, D), lambda b: (b, 0, 0)),
        out_shape=jax.ShapeDtypeStruct((BATCH, SEQ, D), jnp.float32),
    )(xi, si, comb)


def kernel(x, seg, tok_table, pos_table, seg_table):
    comb = _build_comb(tok_table, pos_table, seg_table)
    xi = x.astype(jnp.int8).reshape(NTOK, 1)
    si = seg.astype(jnp.int8).reshape(NTOK, 1)
    return _expand(xi, si, comb)
